# baseline (speedup 1.0000x reference)
import functools

import jax
import jax.numpy as jnp
from jax import lax
from jax.experimental import pallas as pl
from jax.experimental.pallas import tpu as pltpu
from jax.experimental.pallas import tpu_sc as plsc

NC = 2
NS = 16
NW = NC * NS
LANES = 16
CHUNK = 128
NBUF = 2
NPHASE = 2
HALF = 64


def _sc_partials(xp, srcm, dstm, n_pad, rpt):
    d = xp.shape[1] * 2
    nch = srcm.shape[1]
    rowb = CHUNK

    mesh = plsc.VectorSubcoreMesh(core_axis_name="c", subcore_axis_name="s")

    @functools.partial(
        pl.kernel,
        out_type=jax.ShapeDtypeStruct((NC, n_pad, d), jnp.float32),
        mesh=mesh,
        compiler_params=pltpu.CompilerParams(use_tc_tiling_on_sc=False),
        scratch_types=[
            pltpu.VMEM_SHARED((n_pad, d), jnp.float32),
            pltpu.VMEM((nch, CHUNK), jnp.int32),
            pltpu.VMEM((2 * nch, HALF), jnp.int32),
            pltpu.VMEM((NBUF, rowb, d // 2), jnp.int32),
            pltpu.VMEM((HALF, d), jnp.float32),
            pltpu.SemaphoreType.DMA,
            pltpu.SemaphoreType.DMA,
        ],
    )
    def k(xp_hbm, srcm_hbm, dstm_hbm, out_hbm, acc, src_v, dst_v, braw_v,
          rows_v, sem0, sem1):
        cid = lax.axis_index("c")
        sid = lax.axis_index("s")
        wid = sid * NC + cid

        zero16 = jnp.zeros((LANES,), jnp.float32)

        @pl.loop(0, HALF)
        def _zero_rows(r):
            for c in range(d // LANES):
                rows_v[r, pl.ds(c * LANES, LANES)] = zero16

        base = sid * rpt
        for q in range(rpt // HALF):
            pltpu.sync_copy(rows_v, acc.at[pl.ds(base + q * HALF, HALF)])
        plsc.subcore_barrier()

        sems = (sem0, sem1)
        for h in range(NPHASE):
            pltpu.sync_copy(srcm_hbm.at[wid * NPHASE + h], src_v)
            pltpu.sync_copy(dstm_hbm.at[wid * NPHASE + h], dst_v)
            for b in range(NBUF):
                pltpu.async_copy(xp_hbm.at[src_v.at[b]], braw_v.at[b], sems[b])

            @pl.loop(0, nch // NBUF)
            def _ring(g):
                for b in range(NBUF):
                    j = g * NBUF + b
                    bbuf = braw_v.at[b]
                    pltpu.make_async_copy(
                        xp_hbm.at[src_v.at[j]], bbuf, sems[b]).wait()

                    for s in range(rowb // HALF):

                        @pl.loop(0, HALF // 8)
                        def _conv(r8):
                            for r0 in range(8):
                                r = r8 * 8 + r0
                                for c in range(d // 32):
                                    v = bbuf[s * HALF + r,
                                             pl.ds(c * LANES, LANES)]
                                    lo = lax.bitcast_convert_type(
                                        lax.shift_left(v, 16), jnp.float32)
                                    hi = lax.bitcast_convert_type(
                                        lax.bitwise_and(v, jnp.int32(-65536)),
                                        jnp.float32)
                                    rows_v[r, pl.ds(c * 32, LANES)] = lo
                                    rows_v[r, pl.ds(c * 32 + LANES, LANES)] = hi

                        pltpu.sync_copy(
                            rows_v,
                            acc.at[dst_v.at[j * (rowb // HALF) + s]],
                            add=True)

                    @pl.when(j + NBUF < nch)
                    def _fire():
                        pltpu.async_copy(
                            xp_hbm.at[src_v.at[j + NBUF]], braw_v.at[b], sems[b])

        plsc.subcore_barrier()

        pltpu.sync_copy(acc.at[pl.ds(base, rpt)], out_hbm.at[cid].at[pl.ds(base, rpt)])

    return k


def _combine_body(x_ref, p_ref, o_ref, *, scale):
    o_ref[...] = jnp.maximum((x_ref[...] + p_ref[0] + p_ref[1]) * scale, 0.0)


def kernel(x, edge_index, edge_weights, W_w, b_w, att):
    n, d = x.shape
    e = edge_index.shape[1]
    out_dim = att.shape[1]

    src = edge_index[0].astype(jnp.int32)
    dst = edge_index[1].astype(jnp.int32)

    blk_e = NW * CHUNK * NBUF * NPHASE
    ept = -(-e // blk_e) * CHUNK * NBUF * NPHASE
    pad = NW * ept - e

    rpt = -(-(-(-n // NS)) // HALF) * HALF
    n_pad = NS * rpt
    assert n_pad > n

    src_p = jnp.concatenate([src, jnp.zeros((pad,), jnp.int32)])
    dst_p = jnp.concatenate([dst, n + (jnp.arange(pad, dtype=jnp.int32) % (n_pad - n))])
    srcm = src_p.reshape(NW * NPHASE, ept // (NPHASE * CHUNK), CHUNK)
    dstm = dst_p.reshape(NW * NPHASE, ept // (NPHASE * HALF), HALF)

    xp = (
        x.reshape(n, d // 32, 2, LANES)
        .swapaxes(2, 3)
        .reshape(n, d // 2, 2)
        .astype(jnp.bfloat16)
    )
    xp = lax.bitcast_convert_type(xp, jnp.int32)

    partials = _sc_partials(xp, srcm, dstm, n_pad, rpt)(xp, srcm, dstm)

    blk = 1000
    out = pl.pallas_call(
        functools.partial(_combine_body, scale=1.0 / out_dim),
        out_shape=jax.ShapeDtypeStruct((n, d), jnp.float32),
        grid=(n // blk,),
        in_specs=[
            pl.BlockSpec((blk, d), lambda i: (i, 0)),
            pl.BlockSpec((NC, blk, d), lambda i: (0, i, 0)),
        ],
        out_specs=pl.BlockSpec((blk, d), lambda i: (i, 0)),
    )(x, partials)
    return out


# speedup vs baseline: 1.1265x; 1.1265x over previous
import functools

import jax
import jax.numpy as jnp
from jax import lax
from jax.experimental import pallas as pl
from jax.experimental.pallas import tpu as pltpu
from jax.experimental.pallas import tpu_sc as plsc

NC = 2
NS = 16
NW = NC * NS
LANES = 16
CHUNK = 32
NBUF = 10
NPHASE = 1


def _sc_partials(xp, srcm, dstm, n_pad, rpt):
    d = xp.shape[1] * 2
    nch = srcm.shape[1]
    rowb = CHUNK

    mesh = plsc.VectorSubcoreMesh(core_axis_name="c", subcore_axis_name="s")

    @functools.partial(
        pl.kernel,
        out_type=jax.ShapeDtypeStruct((NC, n_pad, d), jnp.float32),
        mesh=mesh,
        compiler_params=pltpu.CompilerParams(use_tc_tiling_on_sc=False),
        scratch_types=[
            pltpu.VMEM_SHARED((n_pad, d), jnp.float32),
            pltpu.VMEM((nch, CHUNK), jnp.int32),
            pltpu.VMEM((nch, CHUNK), jnp.int32),
            pltpu.VMEM((NBUF, rowb, d // 2), jnp.int32),
            pltpu.VMEM((2, rowb, d), jnp.float32),
            pltpu.SemaphoreType.DMA,
            pltpu.SemaphoreType.DMA,
            pltpu.SemaphoreType.DMA,
            pltpu.SemaphoreType.DMA,
            pltpu.SemaphoreType.DMA,
            pltpu.SemaphoreType.DMA,
            pltpu.SemaphoreType.DMA,
            pltpu.SemaphoreType.DMA,
            pltpu.SemaphoreType.DMA,
            pltpu.SemaphoreType.DMA,
            pltpu.SemaphoreType.DMA,
            pltpu.SemaphoreType.DMA,
        ],
    )
    def k(xp_hbm, srcm_hbm, dstm_hbm, out_hbm, acc, src_v, dst_v, braw_v,
          rows_v, sem0, sem1, sem2, sem3, sem4, sem5, sem6, sem7, sem8, sem9,
          ssem0, ssem1):
        cid = lax.axis_index("c")
        sid = lax.axis_index("s")
        wid = sid * NC + cid

        zero16 = jnp.zeros((LANES,), jnp.float32)

        @pl.loop(0, rowb)
        def _zero_rows(r):
            for c in range(d // LANES):
                rows_v[0, r, pl.ds(c * LANES, LANES)] = zero16

        base = sid * rpt
        for q in range(rpt // rowb):
            pltpu.sync_copy(rows_v.at[0], acc.at[pl.ds(base + q * rowb, rowb)])
        plsc.subcore_barrier()

        sems = (sem0, sem1, sem2, sem3, sem4, sem5, sem6, sem7, sem8, sem9)
        for h in range(NPHASE):
            pltpu.sync_copy(srcm_hbm.at[wid * NPHASE + h], src_v)
            pltpu.sync_copy(dstm_hbm.at[wid * NPHASE + h], dst_v)
            for b in range(NBUF):
                pltpu.async_copy(xp_hbm.at[src_v.at[b]], braw_v.at[b], sems[b])

            ssems = (ssem0, ssem1)

            @pl.loop(0, nch // NBUF)
            def _ring(g):
                for b in range(NBUF):
                    j = g * NBUF + b
                    bbuf = braw_v.at[b]
                    fbuf = rows_v.at[b % 2]
                    pltpu.make_async_copy(
                        xp_hbm.at[src_v.at[j]], bbuf, sems[b]).wait()

                    @pl.when(j >= 2)
                    def _drain():
                        pltpu.make_async_copy(
                            fbuf, acc.at[dst_v.at[j - 2]], ssems[b % 2]).wait()

                    @pl.loop(0, rowb // 4)
                    def _conv(r8):
                        for r0 in range(4):
                            r = r8 * 4 + r0
                            for c in range(d // 32):
                                v = bbuf[r, pl.ds(c * LANES, LANES)]
                                lo = lax.bitcast_convert_type(
                                    lax.shift_left(v, 16), jnp.float32)
                                hi = lax.bitcast_convert_type(
                                    lax.bitwise_and(v, jnp.int32(-65536)),
                                    jnp.float32)
                                fbuf[r, pl.ds(c * 32, LANES)] = lo
                                fbuf[r, pl.ds(c * 32 + LANES, LANES)] = hi

                    pltpu.async_copy(fbuf, acc.at[dst_v.at[j]], ssems[b % 2],
                                     add=True)

                    @pl.when(j + NBUF < nch)
                    def _fire():
                        pltpu.async_copy(
                            xp_hbm.at[src_v.at[j + NBUF]], braw_v.at[b], sems[b])

            for b in range(2):
                jj = nch - 2 + b
                pltpu.make_async_copy(
                    rows_v.at[jj % 2], acc.at[dst_v.at[jj]], ssems[jj % 2]).wait()

        plsc.subcore_barrier()

        pltpu.sync_copy(acc.at[pl.ds(base, rpt)], out_hbm.at[cid].at[pl.ds(base, rpt)])

    return k


def _combine_body(x_ref, p_ref, o_ref, *, scale):
    o_ref[...] = jnp.maximum((x_ref[...] + p_ref[0] + p_ref[1]) * scale, 0.0)


def kernel(x, edge_index, edge_weights, W_w, b_w, att):
    n, d = x.shape
    e = edge_index.shape[1]
    out_dim = att.shape[1]

    src = edge_index[0].astype(jnp.int32)
    dst = edge_index[1].astype(jnp.int32)

    blk_e = NW * CHUNK * NBUF * NPHASE
    ept = -(-e // blk_e) * CHUNK * NBUF * NPHASE
    pad = NW * ept - e

    rpt = -(-(-(-n // NS)) // CHUNK) * CHUNK
    n_pad = NS * rpt
    assert n_pad > n

    src_p = jnp.concatenate([src, jnp.zeros((pad,), jnp.int32)])
    dst_p = jnp.concatenate([dst, n + (jnp.arange(pad, dtype=jnp.int32) % (n_pad - n))])
    srcm = src_p.reshape(NW * NPHASE, ept // (NPHASE * CHUNK), CHUNK)
    dstm = dst_p.reshape(NW * NPHASE, ept // (NPHASE * CHUNK), CHUNK)

    xp = (
        x.reshape(n, d // 32, 2, LANES)
        .swapaxes(2, 3)
        .reshape(n, d // 2, 2)
        .astype(jnp.bfloat16)
    )
    xp = lax.bitcast_convert_type(xp, jnp.int32)

    partials = _sc_partials(xp, srcm, dstm, n_pad, rpt)(xp, srcm, dstm)

    blk = 1000
    out = pl.pallas_call(
        functools.partial(_combine_body, scale=1.0 / out_dim),
        out_shape=jax.ShapeDtypeStruct((n, d), jnp.float32),
        grid=(n // blk,),
        in_specs=[
            pl.BlockSpec((blk, d), lambda i: (i, 0)),
            pl.BlockSpec((NC, blk, d), lambda i: (0, i, 0)),
        ],
        out_specs=pl.BlockSpec((blk, d), lambda i: (i, 0)),
    )(x, partials)
    return out
